# baseline (speedup 1.0000x reference)
import functools

import jax
import jax.numpy as jnp
from jax import lax
from jax.experimental import pallas as pl
from jax.experimental.pallas import tpu as pltpu, tpu_sc as plsc

_B = 16384
_L = 16


def _sc_body(x0_hbm, x1_hbm, params_hbm, out_hbm,
             params_v, tab0_v, tab1_v, xv, outv, sem0, sem1, nc):
    wid = lax.axis_index("s") * nc + lax.axis_index("c")
    rows = outv.shape[0]
    base = wid * rows

    cp0 = pltpu.async_copy(x0_hbm.at[pl.ds(base, rows)],
                           xv.at[pl.ds(0, rows)], sem0)
    cp1 = pltpu.async_copy(x1_hbm.at[pl.ds(base, rows)],
                           xv.at[pl.ds(rows, rows)], sem1)
    pltpu.sync_copy(params_hbm, params_v)

    lanes = lax.broadcasted_iota(jnp.int32, (_L,), 0)
    v4 = jnp.minimum(lanes, 6) * 4

    def build(j, ts):
        t0, t1 = ts
        ej = plsc.load_gather(params_v, [v4 + j])
        w0 = plsc.load_gather(params_v, [jnp.full((_L,), 28, jnp.int32) + j])
        w1 = plsc.load_gather(params_v, [jnp.full((_L,), 32, jnp.int32) + j])
        return (t0 + ej * w0, t1 + ej * w1)

    t0 = plsc.load_gather(params_v, [jnp.full((_L,), 36, jnp.int32)])
    t0, t1 = lax.fori_loop(0, 4, build, (t0, jnp.zeros((_L,), jnp.float32)))
    tab0_v[...] = t0
    tab1_v[...] = t1

    cp0.wait()
    cp1.wait()

    def step(r, carry):
        i0 = xv[pl.ds(r * _L, _L)]
        i1 = xv[pl.ds(rows + r * _L, _L)]
        y = plsc.load_gather(tab0_v, [i0]) + plsc.load_gather(tab1_v, [i1])
        outv[pl.ds(r * _L, _L)] = y
        return carry

    lax.fori_loop(0, rows // _L, step, 0, unroll=4)

    pltpu.sync_copy(outv, out_hbm.at[pl.ds(base, rows)])


def kernel(x, emb, W, b):
    info = plsc.get_sparse_core_info()
    nc, ns = info.num_cores, info.num_subcores
    nw = nc * ns
    rows = _B // nw

    x32 = x.astype(jnp.int32)
    x0 = x32[:, 0, 0]
    x1 = x32[:, 1, 0]
    params = jnp.concatenate(
        [emb.reshape(-1), W.reshape(-1), b]).astype(jnp.float32)

    nc, ns = 1, 8
    nw = nc * ns
    rows = _B // nw
    mesh = plsc.VectorSubcoreMesh(core_axis_name="c", subcore_axis_name="s",
                                  num_cores=nc, num_subcores=ns)
    run = pl.kernel(
        functools.partial(_sc_body, nc=nc),
        mesh=mesh,
        compiler_params=pltpu.CompilerParams(needs_layout_passes=False),
        out_type=jax.ShapeDtypeStruct((_B,), jnp.float32),
        scratch_types=[
            pltpu.VMEM((37,), jnp.float32),
            pltpu.VMEM((_L,), jnp.float32),
            pltpu.VMEM((_L,), jnp.float32),
            pltpu.VMEM((2 * rows,), jnp.int32),
            pltpu.VMEM((rows,), jnp.float32),
            pltpu.SemaphoreType.DMA,
            pltpu.SemaphoreType.DMA,
        ],
    )
    out = run(x0, x1, params)
    return out.reshape(_B, 1)


# speedup vs baseline: 1.0160x; 1.0160x over previous
import functools

import jax
import jax.numpy as jnp
from jax import lax
from jax.experimental import pallas as pl
from jax.experimental.pallas import tpu as pltpu, tpu_sc as plsc

_B = 16384
_L = 16


def _sc_body(x0_hbm, x1_hbm, params_hbm, out_hbm,
             params_v, tab0_v, tab1_v, xv, outv, sem0, sem1, nc):
    wid = lax.axis_index("s") * nc + lax.axis_index("c")
    rows = outv.shape[0]
    base = wid * rows

    cp0 = pltpu.async_copy(x0_hbm.at[pl.ds(base, rows)],
                           xv.at[pl.ds(0, rows)], sem0)
    cp1 = pltpu.async_copy(x1_hbm.at[pl.ds(base, rows)],
                           xv.at[pl.ds(rows, rows)], sem1)
    pltpu.sync_copy(params_hbm, params_v)

    lanes = lax.broadcasted_iota(jnp.int32, (_L,), 0)
    v4 = jnp.minimum(lanes, 6) * 4

    def build(j, ts):
        t0, t1 = ts
        ej = plsc.load_gather(params_v, [v4 + j])
        w0 = plsc.load_gather(params_v, [jnp.full((_L,), 28, jnp.int32) + j])
        w1 = plsc.load_gather(params_v, [jnp.full((_L,), 32, jnp.int32) + j])
        return (t0 + ej * w0, t1 + ej * w1)

    t0 = plsc.load_gather(params_v, [jnp.full((_L,), 36, jnp.int32)])
    t0, t1 = lax.fori_loop(0, 4, build, (t0, jnp.zeros((_L,), jnp.float32)))
    tab0_v[...] = t0
    tab1_v[...] = t1

    cp0.wait()
    cp1.wait()

    def step(r, carry):
        i0 = xv[pl.ds(r * _L, _L)]
        i1 = xv[pl.ds(rows + r * _L, _L)]
        y = plsc.load_gather(tab0_v, [i0]) + plsc.load_gather(tab1_v, [i1])
        outv[pl.ds(r * _L, _L)] = y
        return carry

    half = rows // 2
    lax.fori_loop(0, half // _L, step, 0, unroll=4)
    cp_out = pltpu.async_copy(outv.at[pl.ds(0, half)],
                              out_hbm.at[pl.ds(base, half)], sem0)
    lax.fori_loop(half // _L, rows // _L, step, 0, unroll=4)
    cp_out.wait()
    pltpu.sync_copy(outv.at[pl.ds(half, half)],
                    out_hbm.at[pl.ds(base + half, half)])


def kernel(x, emb, W, b):
    info = plsc.get_sparse_core_info()
    nc, ns = info.num_cores, info.num_subcores
    nw = nc * ns
    rows = _B // nw

    x32 = x.astype(jnp.int32)
    x0 = x32[:, 0, 0]
    x1 = x32[:, 1, 0]
    params = jnp.concatenate(
        [emb.reshape(-1), W.reshape(-1), b]).astype(jnp.float32)

    nc, ns = 1, 16
    nw = nc * ns
    rows = _B // nw
    mesh = plsc.VectorSubcoreMesh(core_axis_name="c", subcore_axis_name="s",
                                  num_cores=nc, num_subcores=ns)
    run = pl.kernel(
        functools.partial(_sc_body, nc=nc),
        mesh=mesh,
        compiler_params=pltpu.CompilerParams(needs_layout_passes=False),
        out_type=jax.ShapeDtypeStruct((_B,), jnp.float32),
        scratch_types=[
            pltpu.VMEM((37,), jnp.float32),
            pltpu.VMEM((_L,), jnp.float32),
            pltpu.VMEM((_L,), jnp.float32),
            pltpu.VMEM((2 * rows,), jnp.int32),
            pltpu.VMEM((rows,), jnp.float32),
            pltpu.SemaphoreType.DMA,
            pltpu.SemaphoreType.DMA,
        ],
    )
    out = run(x0, x1, params)
    return out.reshape(_B, 1)


# speedup vs baseline: 1.0175x; 1.0014x over previous
import functools

import jax
import jax.numpy as jnp
from jax import lax
from jax.experimental import pallas as pl
from jax.experimental.pallas import tpu as pltpu, tpu_sc as plsc

_B = 16384
_L = 16
_NS = 16


def _sc_body(x0_hbm, x1_hbm, params_hbm, out_hbm,
             params_v, tab0_v, tab1_v, xv, outv, sem0, sem1, nc):
    wid = lax.axis_index("s") * nc + lax.axis_index("c")
    rows = outv.shape[0]
    base = wid * rows

    cp0 = pltpu.async_copy(x0_hbm.at[pl.ds(base, rows)],
                           xv.at[pl.ds(0, rows)], sem0)
    cp1 = pltpu.async_copy(x1_hbm.at[pl.ds(base, rows)],
                           xv.at[pl.ds(rows, rows)], sem1)
    pltpu.sync_copy(params_hbm, params_v)

    lanes = lax.broadcasted_iota(jnp.int32, (_L,), 0)
    v4 = jnp.minimum(lanes, 6) * 4

    def build(j, ts):
        t0, t1 = ts
        ej = plsc.load_gather(params_v, [v4 + j])
        w0 = plsc.load_gather(params_v, [jnp.full((_L,), 28, jnp.int32) + j])
        w1 = plsc.load_gather(params_v, [jnp.full((_L,), 32, jnp.int32) + j])
        return (t0 + ej * w0, t1 + ej * w1)

    t0 = plsc.load_gather(params_v, [jnp.full((_L,), 36, jnp.int32)])
    t0, t1 = lax.fori_loop(0, 4, build, (t0, jnp.zeros((_L,), jnp.float32)))
    tab0_v[...] = t0
    tab1_v[...] = t1

    cp0.wait()
    cp1.wait()

    def step(r, carry):
        i0 = xv[pl.ds(r * _L, _L)]
        i1 = xv[pl.ds(rows + r * _L, _L)]
        y = plsc.load_gather(tab0_v, [i0]) + plsc.load_gather(tab1_v, [i1])
        outv[pl.ds(r * _L, _L)] = y
        return carry

    half = rows // 2
    lax.fori_loop(0, half // _L, step, 0, unroll=4)
    cp_out = pltpu.async_copy(outv.at[pl.ds(0, half)],
                              out_hbm.at[pl.ds(base, half)], sem0)
    lax.fori_loop(half // _L, rows // _L, step, 0, unroll=4)
    cp_out.wait()
    pltpu.sync_copy(outv.at[pl.ds(half, half)],
                    out_hbm.at[pl.ds(base + half, half)])


def kernel(x, emb, W, b):
    nc = 1
    rows = _B // (nc * _NS)

    x32 = x.astype(jnp.int32)
    x0 = x32[:, 0, 0]
    x1 = x32[:, 1, 0]
    params = jnp.concatenate(
        [emb.reshape(-1), W.reshape(-1), b]).astype(jnp.float32)

    mesh = plsc.VectorSubcoreMesh(core_axis_name="c", subcore_axis_name="s",
                                  num_cores=nc, num_subcores=_NS)
    run = pl.kernel(
        functools.partial(_sc_body, nc=nc),
        mesh=mesh,
        compiler_params=pltpu.CompilerParams(needs_layout_passes=False),
        out_type=jax.ShapeDtypeStruct((_B,), jnp.float32),
        scratch_types=[
            pltpu.VMEM((37,), jnp.float32),
            pltpu.VMEM((_L,), jnp.float32),
            pltpu.VMEM((_L,), jnp.float32),
            pltpu.VMEM((2 * rows,), jnp.int32),
            pltpu.VMEM((rows,), jnp.float32),
            pltpu.SemaphoreType.DMA,
            pltpu.SemaphoreType.DMA,
        ],
    )
    out = run(x0, x1, params)
    return out.reshape(_B, 1)


# speedup vs baseline: 1.0366x; 1.0187x over previous
import functools

import jax
import jax.numpy as jnp
from jax import lax
from jax.experimental import pallas as pl
from jax.experimental.pallas import tpu as pltpu, tpu_sc as plsc

_B = 16384
_L = 16
_NS = 16


def _sc_body(x0_hbm, x1_hbm, params_hbm, out_hbm,
             params_v, tab0_v, tab1_v, xv, outv, sem0, sem1, nc):
    wid = lax.axis_index("s") * nc + lax.axis_index("c")
    rows = outv.shape[0]
    base = wid * rows

    cp0 = pltpu.async_copy(x0_hbm.at[pl.ds(base, rows)],
                           xv.at[pl.ds(0, rows)], sem0)
    cp1 = pltpu.async_copy(x1_hbm.at[pl.ds(base, rows)],
                           xv.at[pl.ds(rows, rows)], sem1)
    pltpu.sync_copy(params_hbm, params_v)

    lanes = lax.broadcasted_iota(jnp.int32, (_L,), 0)
    v4 = jnp.minimum(lanes, 6) * 4

    def build(j, ts):
        t0, t1 = ts
        ej = plsc.load_gather(params_v, [v4 + j])
        w0 = plsc.load_gather(params_v, [jnp.full((_L,), 28, jnp.int32) + j])
        w1 = plsc.load_gather(params_v, [jnp.full((_L,), 32, jnp.int32) + j])
        return (t0 + ej * w0, t1 + ej * w1)

    t0 = plsc.load_gather(params_v, [jnp.full((_L,), 36, jnp.int32)])
    t0, t1 = lax.fori_loop(0, 4, build, (t0, jnp.zeros((_L,), jnp.float32)))
    tab0_v[...] = t0
    tab1_v[...] = t1

    cp0.wait()
    cp1.wait()

    half = rows // 2

    @plsc.parallel_loop(0, half, step=_L, unroll=4)
    def _first(r):
        i0 = xv[pl.ds(r, _L)]
        i1 = xv[pl.ds(rows + r, _L)]
        y = plsc.load_gather(tab0_v, [i0]) + plsc.load_gather(tab1_v, [i1])
        outv[pl.ds(r, _L)] = y

    cp_out = pltpu.async_copy(outv.at[pl.ds(0, half)],
                              out_hbm.at[pl.ds(base, half)], sem0)

    @plsc.parallel_loop(half, rows, step=_L, unroll=4)
    def _second(r):
        i0 = xv[pl.ds(r, _L)]
        i1 = xv[pl.ds(rows + r, _L)]
        y = plsc.load_gather(tab0_v, [i0]) + plsc.load_gather(tab1_v, [i1])
        outv[pl.ds(r, _L)] = y

    cp_out.wait()
    pltpu.sync_copy(outv.at[pl.ds(half, half)],
                    out_hbm.at[pl.ds(base + half, half)])


def kernel(x, emb, W, b):
    nc = 1
    rows = _B // (nc * _NS)

    x32 = x.astype(jnp.int32)
    x0 = x32[:, 0, 0]
    x1 = x32[:, 1, 0]
    params = jnp.concatenate(
        [emb.reshape(-1), W.reshape(-1), b]).astype(jnp.float32)

    mesh = plsc.VectorSubcoreMesh(core_axis_name="c", subcore_axis_name="s",
                                  num_cores=nc, num_subcores=_NS)
    run = pl.kernel(
        functools.partial(_sc_body, nc=nc),
        mesh=mesh,
        compiler_params=pltpu.CompilerParams(needs_layout_passes=False),
        out_type=jax.ShapeDtypeStruct((_B,), jnp.float32),
        scratch_types=[
            pltpu.VMEM((37,), jnp.float32),
            pltpu.VMEM((_L,), jnp.float32),
            pltpu.VMEM((_L,), jnp.float32),
            pltpu.VMEM((2 * rows,), jnp.int32),
            pltpu.VMEM((rows,), jnp.float32),
            pltpu.SemaphoreType.DMA,
            pltpu.SemaphoreType.DMA,
        ],
    )
    out = run(x0, x1, params)
    return out.reshape(_B, 1)


# speedup vs baseline: 1.0807x; 1.0426x over previous
import functools

import jax
import jax.numpy as jnp
from jax import lax
from jax.experimental import pallas as pl
from jax.experimental.pallas import tpu as pltpu, tpu_sc as plsc

_B = 16384
_L = 16
_NS = 16
_NP = 37


def _f32(v):
    return plsc.bitcast(v, jnp.float32)


def _sc_body(xall_hbm, out_hbm, params_v, tab0_v, tab1_v, xv, outv,
             sem0, sem1, nc):
    wid = lax.axis_index("s") * nc + lax.axis_index("c")
    rows = outv.shape[0]
    base = wid * rows

    cp0 = pltpu.async_copy(xall_hbm.at[pl.ds(base, rows)],
                           xv.at[pl.ds(0, rows)], sem0)
    cp1 = pltpu.async_copy(xall_hbm.at[pl.ds(_B + base, rows)],
                           xv.at[pl.ds(rows, rows)], sem1)
    pltpu.sync_copy(xall_hbm.at[pl.ds(2 * _B, _NP)], params_v)

    lanes = lax.broadcasted_iota(jnp.int32, (_L,), 0)
    v4 = jnp.minimum(lanes, 6) * 4

    def build(j, ts):
        t0, t1 = ts
        ej = _f32(plsc.load_gather(params_v, [v4 + j]))
        w0 = _f32(plsc.load_gather(params_v,
                                   [jnp.full((_L,), 28, jnp.int32) + j]))
        w1 = _f32(plsc.load_gather(params_v,
                                   [jnp.full((_L,), 32, jnp.int32) + j]))
        return (t0 + ej * w0, t1 + ej * w1)

    t0 = _f32(plsc.load_gather(params_v, [jnp.full((_L,), 36, jnp.int32)]))
    t0, t1 = lax.fori_loop(0, 4, build, (t0, jnp.zeros((_L,), jnp.float32)))
    tab0_v[...] = t0
    tab1_v[...] = t1

    cp0.wait()
    cp1.wait()

    half = rows // 2

    @plsc.parallel_loop(0, half, step=_L, unroll=4)
    def _first(r):
        i0 = xv[pl.ds(r, _L)]
        i1 = xv[pl.ds(rows + r, _L)]
        y = plsc.load_gather(tab0_v, [i0]) + plsc.load_gather(tab1_v, [i1])
        outv[pl.ds(r, _L)] = y

    cp_out = pltpu.async_copy(outv.at[pl.ds(0, half)],
                              out_hbm.at[pl.ds(base, half)], sem0)

    @plsc.parallel_loop(half, rows, step=_L, unroll=4)
    def _second(r):
        i0 = xv[pl.ds(r, _L)]
        i1 = xv[pl.ds(rows + r, _L)]
        y = plsc.load_gather(tab0_v, [i0]) + plsc.load_gather(tab1_v, [i1])
        outv[pl.ds(r, _L)] = y

    cp_out.wait()
    pltpu.sync_copy(outv.at[pl.ds(half, half)],
                    out_hbm.at[pl.ds(base + half, half)])


def kernel(x, emb, W, b):
    nc = 1
    rows = _B // (nc * _NS)

    x32 = x.astype(jnp.int32)
    params = jnp.concatenate(
        [emb.reshape(-1), W.reshape(-1), b]).astype(jnp.float32)
    xall = jnp.concatenate([
        x32[:, 0, 0], x32[:, 1, 0],
        lax.bitcast_convert_type(params, jnp.int32),
    ])

    mesh = plsc.VectorSubcoreMesh(core_axis_name="c", subcore_axis_name="s",
                                  num_cores=nc, num_subcores=_NS)
    run = pl.kernel(
        functools.partial(_sc_body, nc=nc),
        mesh=mesh,
        compiler_params=pltpu.CompilerParams(needs_layout_passes=False),
        out_type=jax.ShapeDtypeStruct((_B,), jnp.float32),
        scratch_types=[
            pltpu.VMEM((_NP,), jnp.int32),
            pltpu.VMEM((_L,), jnp.float32),
            pltpu.VMEM((_L,), jnp.float32),
            pltpu.VMEM((2 * rows,), jnp.int32),
            pltpu.VMEM((rows,), jnp.float32),
            pltpu.SemaphoreType.DMA,
            pltpu.SemaphoreType.DMA,
        ],
    )
    out = run(xall)
    return out.reshape(_B, 1)
